# baseline (device time: 44096 ns/iter reference)
import jax
import jax.numpy as jnp
from jax import lax
from jax.experimental import pallas as pl
from jax.experimental.pallas import tpu as pltpu

N_DEV = 32
NZ = 4
NQ = 8
M = 1024
N = 1024
HC = N // 2
QROWS = M // NZ
GROWS = M // NQ
SROWS = 32


def kernel(x, w_mat):
    def body(x_ref, w_ref, out_ref, partial_ref,
             a1_rcv, a2_pool, a2_rcv, a3_src, a3_rcv, a4_src, a4_rcv,
             b1_rcv, b2_pool, b2_rcv, b3_src, b3_rcv, b4_src, b4_rcv,
             a1_ss, a1_rs, a2_ss, a2_rs, a3_ss, a3_rs, a4_ss, a4_rs,
             b1_ss, b1_rs, b2_ss, b2_rs, b3_ss, b3_rs, b4_ss, b4_rs):
        my = lax.axis_index("i")
        z = my // NQ
        q = my % NQ

        ca = pl.ds(0, HC)
        cb = pl.ds(HC, HC)

        def zpeer(k):
            return ((z + k) % NZ) * NQ + q

        def qpeer(j):
            return z * NQ + (q + j) % NQ

        xbf = x_ref[:, :].astype(jnp.bfloat16)
        partial_ref[:, ca] = lax.dot_general(
            xbf, w_ref[:, ca].astype(jnp.bfloat16),
            (((1,), (0,)), ((), ())),
            preferred_element_type=jnp.float32,
        ).astype(jnp.bfloat16)

        barrier = pltpu.get_barrier_semaphore()
        for k in range(1, NZ):
            pl.semaphore_signal(barrier, inc=1, device_id=(zpeer(k),),
                                device_id_type=pl.DeviceIdType.MESH)
        for j in range(1, NQ):
            pl.semaphore_signal(barrier, inc=1, device_id=(qpeer(j),),
                                device_id_type=pl.DeviceIdType.MESH)
        pl.semaphore_wait(barrier, (NZ - 1) + (NQ - 1))

        def copy(src, dst, ss, rs, dev):
            r = pltpu.make_async_remote_copy(
                src_ref=src, dst_ref=dst, send_sem=ss, recv_sem=rs,
                device_id=(dev,), device_id_type=pl.DeviceIdType.MESH)
            r.start()
            return r

        a1 = [None] * NZ
        for k in range(1, NZ):
            zz = (z + k) % NZ
            a1[k] = copy(partial_ref.at[pl.ds(zz * QROWS, QROWS), ca],
                         a1_rcv.at[k - 1], a1_ss.at[k - 1], a1_rs.at[k - 1],
                         zpeer(k))

        partial_ref[:, cb] = lax.dot_general(
            xbf, w_ref[:, cb].astype(jnp.bfloat16),
            (((1,), (0,)), ((), ())),
            preferred_element_type=jnp.float32,
        ).astype(jnp.bfloat16)

        b1 = [None] * NQ
        for j in range(1, NQ):
            qq = (q + j) % NQ
            b1[j] = copy(partial_ref.at[pl.ds(qq * GROWS, GROWS), cb],
                         b1_rcv.at[j - 1], b1_ss.at[j - 1], b1_rs.at[j - 1],
                         qpeer(j))

        accA1 = partial_ref[pl.ds(z * QROWS, QROWS), ca].astype(jnp.float32)
        for k in range(1, NZ):
            a1[k].wait_recv()
            accA1 = accA1 + a1_rcv[k - 1, :, :].astype(jnp.float32)
        a2_pool[:, :] = accA1.astype(jnp.bfloat16)
        a2 = [None] * NQ
        for j in range(1, NQ):
            qq = (q + j) % NQ
            a2[j] = copy(a2_pool.at[pl.ds(qq * SROWS, SROWS)],
                         a2_rcv.at[j - 1], a2_ss.at[j - 1], a2_rs.at[j - 1],
                         qpeer(j))

        accB1 = partial_ref[pl.ds(q * GROWS, GROWS), cb].astype(jnp.float32)
        for j in range(1, NQ):
            b1[j].wait_recv()
            accB1 = accB1 + b1_rcv[j - 1, :, :].astype(jnp.float32)
        b2_pool[:, :] = accB1.astype(jnp.bfloat16)
        b2 = [None] * NZ
        for k in range(1, NZ):
            zz = (z + k) % NZ
            b2[k] = copy(b2_pool.at[pl.ds(zz * SROWS, SROWS)],
                         b2_rcv.at[k - 1], b2_ss.at[k - 1], b2_rs.at[k - 1],
                         zpeer(k))

        accA = a2_pool[pl.ds(q * SROWS, SROWS), :].astype(jnp.float32)
        for j in range(1, NQ):
            a2[j].wait_recv()
            accA = accA + a2_rcv[j - 1, :, :].astype(jnp.float32)
        a3_src[:, :] = accA.astype(jnp.bfloat16)
        a3 = [None] * NQ
        for j in range(1, NQ):
            a3[j] = copy(a3_src, a3_rcv.at[j - 1],
                         a3_ss.at[j - 1], a3_rs.at[j - 1], qpeer(j))
        out_ref[pl.ds(z * QROWS + q * SROWS, SROWS), ca] = jnp.maximum(accA, 0.0)
        a4_src[pl.ds(q * SROWS, SROWS), :] = a3_src[:, :]

        accB = b2_pool[pl.ds(z * SROWS, SROWS), :].astype(jnp.float32)
        for k in range(1, NZ):
            b2[k].wait_recv()
            accB = accB + b2_rcv[k - 1, :, :].astype(jnp.float32)
        b3_src[:, :] = accB.astype(jnp.bfloat16)
        b3 = [None] * NZ
        for k in range(1, NZ):
            b3[k] = copy(b3_src, b3_rcv.at[k - 1],
                         b3_ss.at[k - 1], b3_rs.at[k - 1], zpeer(k))
        out_ref[pl.ds(q * GROWS + z * SROWS, SROWS), cb] = jnp.maximum(accB, 0.0)
        b4_src[pl.ds(z * SROWS, SROWS), :] = b3_src[:, :]

        for j in range(1, NQ):
            a3[j].wait_recv()
            qq = (q - j) % NQ
            a4_src[pl.ds(qq * SROWS, SROWS), :] = a3_rcv[j - 1, :, :]
            out_ref[pl.ds(z * QROWS + qq * SROWS, SROWS), ca] = jnp.maximum(
                a3_rcv[j - 1, :, :].astype(jnp.float32), 0.0)
        a4 = [None] * NZ
        for k in range(1, NZ):
            a4[k] = copy(a4_src, a4_rcv.at[k - 1],
                         a4_ss.at[k - 1], a4_rs.at[k - 1], zpeer(k))

        for k in range(1, NZ):
            b3[k].wait_recv()
            zz = (z - k) % NZ
            b4_src[pl.ds(zz * SROWS, SROWS), :] = b3_rcv[k - 1, :, :]
            out_ref[pl.ds(q * GROWS + zz * SROWS, SROWS), cb] = jnp.maximum(
                b3_rcv[k - 1, :, :].astype(jnp.float32), 0.0)
        b4 = [None] * NQ
        for j in range(1, NQ):
            b4[j] = copy(b4_src, b4_rcv.at[j - 1],
                         b4_ss.at[j - 1], b4_rs.at[j - 1], qpeer(j))

        for k in range(1, NZ):
            a4[k].wait_recv()
            zz = (z - k) % NZ
            out_ref[pl.ds(zz * QROWS, QROWS), ca] = jnp.maximum(
                a4_rcv[k - 1, :, :].astype(jnp.float32), 0.0)

        for j in range(1, NQ):
            b4[j].wait_recv()
            qq = (q - j) % NQ
            out_ref[pl.ds(qq * GROWS, GROWS), cb] = jnp.maximum(
                b4_rcv[j - 1, :, :].astype(jnp.float32), 0.0)

        for k in range(1, NZ):
            a1[k].wait_send()
            b2[k].wait_send()
            b3[k].wait_send()
            a4[k].wait_send()
        for j in range(1, NQ):
            b1[j].wait_send()
            a2[j].wait_send()
            a3[j].wait_send()
            b4[j].wait_send()

    bf = jnp.bfloat16
    return pl.pallas_call(
        body,
        out_shape=jax.ShapeDtypeStruct((M, N), jnp.float32),
        in_specs=[
            pl.BlockSpec(memory_space=pltpu.VMEM),
            pl.BlockSpec(memory_space=pltpu.VMEM),
        ],
        out_specs=pl.BlockSpec(memory_space=pltpu.VMEM),
        scratch_shapes=[
            pltpu.VMEM((M, N), bf),
            pltpu.VMEM((NZ - 1, QROWS, HC), bf),
            pltpu.VMEM((QROWS, HC), bf),
            pltpu.VMEM((NQ - 1, SROWS, HC), bf),
            pltpu.VMEM((SROWS, HC), bf),
            pltpu.VMEM((NQ - 1, SROWS, HC), bf),
            pltpu.VMEM((QROWS, HC), bf),
            pltpu.VMEM((NZ - 1, QROWS, HC), bf),
            pltpu.VMEM((NQ - 1, GROWS, HC), bf),
            pltpu.VMEM((GROWS, HC), bf),
            pltpu.VMEM((NZ - 1, SROWS, HC), bf),
            pltpu.VMEM((SROWS, HC), bf),
            pltpu.VMEM((NZ - 1, SROWS, HC), bf),
            pltpu.VMEM((GROWS, HC), bf),
            pltpu.VMEM((NQ - 1, GROWS, HC), bf),
            pltpu.SemaphoreType.DMA((NZ - 1,)),
            pltpu.SemaphoreType.DMA((NZ - 1,)),
            pltpu.SemaphoreType.DMA((NQ - 1,)),
            pltpu.SemaphoreType.DMA((NQ - 1,)),
            pltpu.SemaphoreType.DMA((NQ - 1,)),
            pltpu.SemaphoreType.DMA((NQ - 1,)),
            pltpu.SemaphoreType.DMA((NZ - 1,)),
            pltpu.SemaphoreType.DMA((NZ - 1,)),
            pltpu.SemaphoreType.DMA((NQ - 1,)),
            pltpu.SemaphoreType.DMA((NQ - 1,)),
            pltpu.SemaphoreType.DMA((NZ - 1,)),
            pltpu.SemaphoreType.DMA((NZ - 1,)),
            pltpu.SemaphoreType.DMA((NZ - 1,)),
            pltpu.SemaphoreType.DMA((NZ - 1,)),
            pltpu.SemaphoreType.DMA((NQ - 1,)),
            pltpu.SemaphoreType.DMA((NQ - 1,)),
        ],
        compiler_params=pltpu.CompilerParams(collective_id=0),
    )(x, w_mat)


# device time: 43695 ns/iter; 1.0092x vs baseline; 1.0092x over previous
import jax
import jax.numpy as jnp
from jax import lax
from jax.experimental import pallas as pl
from jax.experimental.pallas import tpu as pltpu

N_DEV = 32
NZ = 4
NQ = 8
NP = 2
M = 1024
N = 1024
HC = N // 2
PC = HC // NP
QROWS = M // NZ
GROWS = M // NQ
SROWS = 32


def kernel(x, w_mat):
    def body(x_ref, w_ref, out_ref, partial_ref,
             a1_rcv, a2_pool, a2_rcv, a3_src, a3_rcv, a4_src, a4_rcv,
             b1_rcv, b2_pool, b2_rcv, b3_src, b3_rcv, b4_src, b4_rcv,
             a1_ss, a1_rs, a2_ss, a2_rs, a3_ss, a3_rs, a4_ss, a4_rs,
             b1_ss, b1_rs, b2_ss, b2_rs, b3_ss, b3_rs, b4_ss, b4_rs):
        my = lax.axis_index("i")
        z = my // NQ
        q = my % NQ

        def acols(p):
            return pl.ds(p * PC, PC)

        def bcols(p):
            return pl.ds(HC + p * PC, PC)

        def zpeer(k):
            return ((z + k) % NZ) * NQ + q

        def qpeer(j):
            return z * NQ + (q + j) % NQ

        xbf = x_ref[:, :].astype(jnp.bfloat16)
        partial_ref[:, pl.ds(0, HC)] = lax.dot_general(
            xbf, w_ref[:, pl.ds(0, HC)].astype(jnp.bfloat16),
            (((1,), (0,)), ((), ())),
            preferred_element_type=jnp.float32,
        ).astype(jnp.bfloat16)

        barrier = pltpu.get_barrier_semaphore()
        for k in range(1, NZ):
            pl.semaphore_signal(barrier, inc=1, device_id=(zpeer(k),),
                                device_id_type=pl.DeviceIdType.MESH)
        for j in range(1, NQ):
            pl.semaphore_signal(barrier, inc=1, device_id=(qpeer(j),),
                                device_id_type=pl.DeviceIdType.MESH)
        pl.semaphore_wait(barrier, (NZ - 1) + (NQ - 1))

        def copy(src, dst, ss, rs, dev):
            r = pltpu.make_async_remote_copy(
                src_ref=src, dst_ref=dst, send_sem=ss, recv_sem=rs,
                device_id=(dev,), device_id_type=pl.DeviceIdType.MESH)
            r.start()
            return r

        a1 = [[None] * NZ for _ in range(NP)]
        a2 = [[None] * NQ for _ in range(NP)]
        a3 = [[None] * NQ for _ in range(NP)]
        a4 = [[None] * NZ for _ in range(NP)]
        b1 = [[None] * NQ for _ in range(NP)]
        b2 = [[None] * NZ for _ in range(NP)]
        b3 = [[None] * NZ for _ in range(NP)]
        b4 = [[None] * NQ for _ in range(NP)]


        def a1_send(p):
            for k in range(1, NZ):
                zz = (z + k) % NZ
                a1[p][k] = copy(
                    partial_ref.at[pl.ds(zz * QROWS, QROWS), acols(p)],
                    a1_rcv.at[p, k - 1], a1_ss.at[p, k - 1],
                    a1_rs.at[p, k - 1], zpeer(k))

        def b1_send(p):
            for j in range(1, NQ):
                qq = (q + j) % NQ
                b1[p][j] = copy(
                    partial_ref.at[pl.ds(qq * GROWS, GROWS), bcols(p)],
                    b1_rcv.at[p, j - 1], b1_ss.at[p, j - 1],
                    b1_rs.at[p, j - 1], qpeer(j))

        def a1_reduce_a2_send(p):
            accA1 = partial_ref[pl.ds(z * QROWS, QROWS), acols(p)].astype(
                jnp.float32)
            for k in range(1, NZ):
                a1[p][k].wait_recv()
                accA1 = accA1 + a1_rcv[p, k - 1, :, :].astype(jnp.float32)
            a2_pool[p, :, :] = accA1.astype(jnp.bfloat16)
            for j in range(1, NQ):
                qq = (q + j) % NQ
                a2[p][j] = copy(
                    a2_pool.at[p, pl.ds(qq * SROWS, SROWS)],
                    a2_rcv.at[p, j - 1], a2_ss.at[p, j - 1],
                    a2_rs.at[p, j - 1], qpeer(j))

        def b1_reduce_b2_send(p):
            accB1 = partial_ref[pl.ds(q * GROWS, GROWS), bcols(p)].astype(
                jnp.float32)
            for j in range(1, NQ):
                b1[p][j].wait_recv()
                accB1 = accB1 + b1_rcv[p, j - 1, :, :].astype(jnp.float32)
            b2_pool[p, :, :] = accB1.astype(jnp.bfloat16)
            for k in range(1, NZ):
                zz = (z + k) % NZ
                b2[p][k] = copy(
                    b2_pool.at[p, pl.ds(zz * SROWS, SROWS)],
                    b2_rcv.at[p, k - 1], b2_ss.at[p, k - 1],
                    b2_rs.at[p, k - 1], zpeer(k))

        def a2_reduce_a3_send(p):
            accA = a2_pool[p, pl.ds(q * SROWS, SROWS), :].astype(jnp.float32)
            for j in range(1, NQ):
                a2[p][j].wait_recv()
                accA = accA + a2_rcv[p, j - 1, :, :].astype(jnp.float32)
            a3_src[p, :, :] = accA.astype(jnp.bfloat16)
            for j in range(1, NQ):
                a3[p][j] = copy(a3_src.at[p], a3_rcv.at[p, j - 1],
                                a3_ss.at[p, j - 1], a3_rs.at[p, j - 1],
                                qpeer(j))
            out_ref[pl.ds(z * QROWS + q * SROWS, SROWS), acols(p)] = (
                jnp.maximum(accA, 0.0))
            a4_src[p, pl.ds(q * SROWS, SROWS), :] = a3_src[p, :, :]

        def b2_reduce_b3_send(p):
            accB = b2_pool[p, pl.ds(z * SROWS, SROWS), :].astype(jnp.float32)
            for k in range(1, NZ):
                b2[p][k].wait_recv()
                accB = accB + b2_rcv[p, k - 1, :, :].astype(jnp.float32)
            b3_src[p, :, :] = accB.astype(jnp.bfloat16)
            for k in range(1, NZ):
                b3[p][k] = copy(b3_src.at[p], b3_rcv.at[p, k - 1],
                                b3_ss.at[p, k - 1], b3_rs.at[p, k - 1],
                                zpeer(k))
            out_ref[pl.ds(q * GROWS + z * SROWS, SROWS), bcols(p)] = (
                jnp.maximum(accB, 0.0))
            b4_src[p, pl.ds(z * SROWS, SROWS), :] = b3_src[p, :, :]

        def a3_recv_a4_send(p):
            for j in range(1, NQ):
                a3[p][j].wait_recv()
                qq = (q - j) % NQ
                a4_src[p, pl.ds(qq * SROWS, SROWS), :] = a3_rcv[p, j - 1, :, :]
                out_ref[pl.ds(z * QROWS + qq * SROWS, SROWS), acols(p)] = (
                    jnp.maximum(a3_rcv[p, j - 1, :, :].astype(jnp.float32),
                                0.0))
            for k in range(1, NZ):
                a4[p][k] = copy(a4_src.at[p], a4_rcv.at[p, k - 1],
                                a4_ss.at[p, k - 1], a4_rs.at[p, k - 1],
                                zpeer(k))

        def b3_recv_b4_send(p):
            for k in range(1, NZ):
                b3[p][k].wait_recv()
                zz = (z - k) % NZ
                b4_src[p, pl.ds(zz * SROWS, SROWS), :] = b3_rcv[p, k - 1, :, :]
                out_ref[pl.ds(q * GROWS + zz * SROWS, SROWS), bcols(p)] = (
                    jnp.maximum(b3_rcv[p, k - 1, :, :].astype(jnp.float32),
                                0.0))
            for j in range(1, NQ):
                b4[p][j] = copy(b4_src.at[p], b4_rcv.at[p, j - 1],
                                b4_ss.at[p, j - 1], b4_rs.at[p, j - 1],
                                qpeer(j))

        def a4_store(p):
            for k in range(1, NZ):
                a4[p][k].wait_recv()
                zz = (z - k) % NZ
                out_ref[pl.ds(zz * QROWS, QROWS), acols(p)] = jnp.maximum(
                    a4_rcv[p, k - 1, :, :].astype(jnp.float32), 0.0)

        def b4_store(p):
            for j in range(1, NQ):
                b4[p][j].wait_recv()
                qq = (q - j) % NQ
                out_ref[pl.ds(qq * GROWS, GROWS), bcols(p)] = jnp.maximum(
                    b4_rcv[p, j - 1, :, :].astype(jnp.float32), 0.0)

        a1_send(0)
        a1_send(1)

        partial_ref[:, pl.ds(HC, HC)] = lax.dot_general(
            xbf, w_ref[:, pl.ds(HC, HC)].astype(jnp.bfloat16),
            (((1,), (0,)), ((), ())),
            preferred_element_type=jnp.float32,
        ).astype(jnp.bfloat16)
        b1_send(0)
        b1_send(1)

        a1_reduce_a2_send(0)
        b1_reduce_b2_send(0)
        a1_reduce_a2_send(1)
        b1_reduce_b2_send(1)

        a2_reduce_a3_send(0)
        b2_reduce_b3_send(0)
        a2_reduce_a3_send(1)
        b2_reduce_b3_send(1)

        a3_recv_a4_send(0)
        b3_recv_b4_send(0)
        a3_recv_a4_send(1)
        b3_recv_b4_send(1)

        a4_store(0)
        b4_store(0)
        a4_store(1)
        b4_store(1)

        for p in range(NP):
            for k in range(1, NZ):
                a1[p][k].wait_send()
                b2[p][k].wait_send()
                b3[p][k].wait_send()
                a4[p][k].wait_send()
            for j in range(1, NQ):
                b1[p][j].wait_send()
                a2[p][j].wait_send()
                a3[p][j].wait_send()
                b4[p][j].wait_send()

    bf = jnp.bfloat16
    return pl.pallas_call(
        body,
        out_shape=jax.ShapeDtypeStruct((M, N), jnp.float32),
        in_specs=[
            pl.BlockSpec(memory_space=pltpu.VMEM),
            pl.BlockSpec(memory_space=pltpu.VMEM),
        ],
        out_specs=pl.BlockSpec(memory_space=pltpu.VMEM),
        scratch_shapes=[
            pltpu.VMEM((M, N), bf),
            pltpu.VMEM((NP, NZ - 1, QROWS, PC), bf),
            pltpu.VMEM((NP, QROWS, PC), bf),
            pltpu.VMEM((NP, NQ - 1, SROWS, PC), bf),
            pltpu.VMEM((NP, SROWS, PC), bf),
            pltpu.VMEM((NP, NQ - 1, SROWS, PC), bf),
            pltpu.VMEM((NP, QROWS, PC), bf),
            pltpu.VMEM((NP, NZ - 1, QROWS, PC), bf),
            pltpu.VMEM((NP, NQ - 1, GROWS, PC), bf),
            pltpu.VMEM((NP, GROWS, PC), bf),
            pltpu.VMEM((NP, NZ - 1, SROWS, PC), bf),
            pltpu.VMEM((NP, SROWS, PC), bf),
            pltpu.VMEM((NP, NZ - 1, SROWS, PC), bf),
            pltpu.VMEM((NP, GROWS, PC), bf),
            pltpu.VMEM((NP, NQ - 1, GROWS, PC), bf),
            pltpu.SemaphoreType.DMA((NP, NZ - 1)),
            pltpu.SemaphoreType.DMA((NP, NZ - 1)),
            pltpu.SemaphoreType.DMA((NP, NQ - 1)),
            pltpu.SemaphoreType.DMA((NP, NQ - 1)),
            pltpu.SemaphoreType.DMA((NP, NQ - 1)),
            pltpu.SemaphoreType.DMA((NP, NQ - 1)),
            pltpu.SemaphoreType.DMA((NP, NZ - 1)),
            pltpu.SemaphoreType.DMA((NP, NZ - 1)),
            pltpu.SemaphoreType.DMA((NP, NQ - 1)),
            pltpu.SemaphoreType.DMA((NP, NQ - 1)),
            pltpu.SemaphoreType.DMA((NP, NZ - 1)),
            pltpu.SemaphoreType.DMA((NP, NZ - 1)),
            pltpu.SemaphoreType.DMA((NP, NZ - 1)),
            pltpu.SemaphoreType.DMA((NP, NZ - 1)),
            pltpu.SemaphoreType.DMA((NP, NQ - 1)),
            pltpu.SemaphoreType.DMA((NP, NQ - 1)),
        ],
        compiler_params=pltpu.CompilerParams(collective_id=0),
    )(x, w_mat)
